# baseline (speedup 1.0000x reference)
import jax
import jax.numpy as jnp
from jax.experimental import pallas as pl

NUM_SAMPLE = 32


def _body(nf_ref, nb_ref, w1_ref, w2_ref, out_ref):
    bn = nf_ref.shape[0]
    d = nf_ref.shape[1]
    nb = nb_ref[...]
    mean = jnp.mean(nb.reshape(bn, NUM_SAMPLE, d), axis=1)
    nf = nf_ref[...]
    top = jax.lax.dot_general(w1_ref[...], nf, (((1,), (1,)), ((), ())),
                              preferred_element_type=jnp.float32)
    bot = jax.lax.dot_general(w2_ref[...], mean, (((1,), (1,)), ((), ())),
                              preferred_element_type=jnp.float32)
    out_ref[...] = jnp.maximum(jnp.concatenate([top, bot], axis=0), 0.0)


def kernel(node_feats, neighbor_feats, node_count, W1, W2):
    n, d = node_feats.shape
    e2 = W1.shape[0]
    BN = 1024
    grid = (pl.cdiv(n, BN),)
    out = pl.pallas_call(
        _body,
        grid=grid,
        in_specs=[
            pl.BlockSpec((BN, d), lambda i: (i, 0)),
            pl.BlockSpec((BN * NUM_SAMPLE, d), lambda i: (i, 0)),
            pl.BlockSpec((e2, d), lambda i: (0, 0)),
            pl.BlockSpec((e2, d), lambda i: (0, 0)),
        ],
        out_specs=pl.BlockSpec((2 * e2, BN), lambda i: (0, i)),
        out_shape=jax.ShapeDtypeStruct((2 * e2, n), jnp.float32),
    )(node_feats, neighbor_feats, W1, W2)
    return out


# speedup vs baseline: 1.0184x; 1.0184x over previous
import jax
import jax.numpy as jnp
from jax.experimental import pallas as pl

NUM_SAMPLE = 32


def _body(nf_ref, nb_ref, w1_ref, w2_ref, out_ref):
    bn = nf_ref.shape[0]
    d = nf_ref.shape[1]
    nb = nb_ref[...]
    mean = jnp.mean(nb.reshape(bn, NUM_SAMPLE, d), axis=1)
    nf = nf_ref[...]
    top = jax.lax.dot_general(w1_ref[...], nf, (((1,), (1,)), ((), ())),
                              preferred_element_type=jnp.float32)
    bot = jax.lax.dot_general(w2_ref[...], mean, (((1,), (1,)), ((), ())),
                              preferred_element_type=jnp.float32)
    out_ref[...] = jnp.maximum(jnp.concatenate([top, bot], axis=0), 0.0)


def kernel(node_feats, neighbor_feats, node_count, W1, W2):
    n, d = node_feats.shape
    e2 = W1.shape[0]
    BN = 512
    grid = (pl.cdiv(n, BN),)
    out = pl.pallas_call(
        _body,
        grid=grid,
        in_specs=[
            pl.BlockSpec((BN, d), lambda i: (i, 0)),
            pl.BlockSpec((BN * NUM_SAMPLE, d), lambda i: (i, 0)),
            pl.BlockSpec((e2, d), lambda i: (0, 0)),
            pl.BlockSpec((e2, d), lambda i: (0, 0)),
        ],
        out_specs=pl.BlockSpec((2 * e2, BN), lambda i: (0, i)),
        out_shape=jax.ShapeDtypeStruct((2 * e2, n), jnp.float32),
    )(node_feats, neighbor_feats, W1, W2)
    return out
